# Initial kernel scaffold; baseline (speedup 1.0000x reference)
import jax
import jax.numpy as jnp
from jax.experimental import pallas as pl


def kernel(x, edge_index, W1, b1, W2, b2, Wself, bself):
    raise NotImplementedError("write your pallas kernel here")


# speedup vs baseline: 3.6910x; 3.6910x over previous
import functools

import jax
import jax.numpy as jnp
from jax import lax
from jax.experimental import pallas as pl
from jax.experimental.pallas import tpu as pltpu
from jax.experimental.pallas import tpu_sc as plsc

N_NODES = 10000
N_EDGES = 320000
D = 128
DA = 144

NC = 1
NS = 16
NW = NC * NS

CHUNK = 128
NCHUNK = 160
E_PAD = NW * NCHUNK * CHUNK
IG = 5
NIG = NCHUNK // IG
NSG = NCHUNK // 10

N_ACC = 10016
ZROWS = N_ACC // NS

BLK = 1000


def _sc_agg_body(xa_hbm, src_hbm, dst_hbm, zeros_hbm, out_hbm,
                 src_idx, dst_idx, rows, acc, *sems):
    gsem = sems[0:2]
    ssem = sems[2:4]
    isem = sems[4:6]
    jsem = sems[6:8]
    sid = lax.axis_index("s")
    wid = sid * NC + lax.axis_index("c")

    pltpu.sync_copy(zeros_hbm, acc.at[pl.ds(sid * ZROWS, ZROWS)])

    def fetch_idx(gi, b):
        pltpu.async_copy(src_hbm.at[wid * NIG + gi], src_idx.at[b], isem[b])
        pltpu.async_copy(dst_hbm.at[wid * NIG + gi], dst_idx.at[b], jsem[b])

    def wait_idx(gi, b):
        pltpu.make_async_copy(src_hbm.at[wid * NIG + gi], src_idx.at[b],
                              isem[b]).wait()
        pltpu.make_async_copy(dst_hbm.at[wid * NIG + gi], dst_idx.at[b],
                              jsem[b]).wait()

    def start_gather(bi, slot, rb):
        pltpu.async_copy(xa_hbm.at[src_idx.at[bi, slot]], rows.at[rb],
                         gsem[rb])

    def wait_gather(bi, slot, rb):
        pltpu.make_async_copy(xa_hbm.at[src_idx.at[bi, slot]], rows.at[rb],
                              gsem[rb]).wait()

    def start_scatter(bi, slot, rb):
        pltpu.async_copy(rows.at[rb], acc.at[dst_idx.at[bi, slot]], ssem[rb],
                         add=True)

    def wait_scatter(bi, slot, rb):
        pltpu.make_async_copy(rows.at[rb], acc.at[dst_idx.at[bi, slot]],
                              ssem[rb]).wait()

    fetch_idx(0, 0)
    wait_idx(0, 0)
    fetch_idx(1, 1)
    plsc.subcore_barrier()
    start_gather(0, 0, 0)
    start_gather(0, 1, 1)
    wait_gather(0, 0, 0)
    start_scatter(0, 0, 0)
    for t in range(2, 10):
        bi, slot, rb = t // 5, t % 5, t % 2
        pb, ps = (t - 1) // 5, (t - 1) % 5
        wait_scatter(bi, slot, rb)
        start_gather(bi, slot, rb)
        wait_gather(pb, ps, 1 - rb)
        start_scatter(pb, ps, 1 - rb)
        if t == 5:
            wait_idx(1, 1)
        if t == 7:
            fetch_idx(2, 0)

    def sg_body(sg, carry):
        g0 = sg * 2
        for t in range(10):
            bi, slot, rb = t // 5, t % 5, t % 2
            pb, ps = (t - 1) // 5, (t - 1) % 5
            if t == 0:
                pb, ps = 1, 4
                wait_idx(g0, 0)
            if t == 5:
                wait_idx(g0 + 1, 1)
            wait_scatter(bi, slot, rb)
            start_gather(bi, slot, rb)
            wait_gather(pb, ps, 1 - rb)
            start_scatter(pb, ps, 1 - rb)
            if t == 2:
                fetch_idx(g0 + 1, 1)
            if t == 7:
                fetch_idx(jnp.minimum(g0 + 2, NIG - 1), 0)
        return carry

    lax.fori_loop(1, NSG, sg_body, 0)

    wait_gather(1, 4, 1)
    start_scatter(1, 4, 1)
    wait_scatter(1, 3, 0)
    wait_scatter(1, 4, 1)
    wait_idx(NIG - 1, 0)

    plsc.subcore_barrier()

    rpt = 624
    r0 = sid * rpt
    pltpu.sync_copy(acc.at[pl.ds(r0, rpt)], out_hbm.at[pl.ds(r0, rpt)])

    @pl.when(sid == NS - 1)
    def _tail():
        t0 = NS * rpt
        pltpu.sync_copy(acc.at[pl.ds(t0, N_NODES - t0)],
                        out_hbm.at[pl.ds(t0, N_NODES - t0)])


_sc_agg = functools.partial(
    pl.kernel,
    out_type=jax.ShapeDtypeStruct((N_NODES, DA), jnp.float32),
    mesh=plsc.VectorSubcoreMesh(core_axis_name="c", subcore_axis_name="s",
                                num_cores=NC),
    scratch_types=[
        pltpu.VMEM((2, IG, CHUNK), jnp.int32),
        pltpu.VMEM((2, IG, CHUNK), jnp.int32),
        pltpu.VMEM((2, CHUNK, DA), jnp.float32),
        pltpu.VMEM_SHARED((N_ACC, DA), jnp.float32),
    ] + [pltpu.SemaphoreType.DMA] * 8,
    compiler_params=pltpu.CompilerParams(use_tc_tiling_on_sc=False),
)(_sc_agg_body)


def _dot_t(a, w):
    return lax.dot_general(a, w, (((1,), (1,)), ((), ())),
                           preferred_element_type=jnp.float32)


def _tc_body(p0, xr, w1, b1r, w2, b2r, ws, bsr, o):
    a = p0[...]
    neigh = a[:, :D]
    deg = a[:, D:D + 1]
    neigh = neigh * (1.0 / jnp.maximum(deg, 1.0))
    h = jnp.maximum(_dot_t(neigh, w1[...]) + b1r[...], 0.0)
    h = _dot_t(h, w2[...]) + b2r[...]
    o[...] = h + _dot_t(xr[...], ws[...]) + bsr[...]


def _tc_mlp(partials, x, W1, b1r, W2, b2r, Wself, bsr):
    nblk = N_NODES // BLK
    return pl.pallas_call(
        _tc_body,
        grid=(nblk,),
        in_specs=[
            pl.BlockSpec((BLK, DA), lambda i: (i, 0)),
            pl.BlockSpec((BLK, D), lambda i: (i, 0)),
            pl.BlockSpec((D, D), lambda i: (0, 0)),
            pl.BlockSpec((1, D), lambda i: (0, 0)),
            pl.BlockSpec((D, D), lambda i: (0, 0)),
            pl.BlockSpec((1, D), lambda i: (0, 0)),
            pl.BlockSpec((D, D), lambda i: (0, 0)),
            pl.BlockSpec((1, D), lambda i: (0, 0)),
        ],
        out_specs=pl.BlockSpec((BLK, D), lambda i: (i, 0)),
        out_shape=jax.ShapeDtypeStruct((N_NODES, D), jnp.float32),
    )(partials, x, W1, b1r, W2, b2r, Wself, bsr)


def kernel(x, edge_index, W1, b1, W2, b2, Wself, bself):
    src = edge_index[0].astype(jnp.int32)
    dst = edge_index[1].astype(jnp.int32)
    pad = E_PAD - N_EDGES
    src_p = jnp.concatenate([src, jnp.zeros((pad,), jnp.int32)])
    dst_p = jnp.concatenate([dst, jnp.full((pad,), N_NODES, jnp.int32)])
    src_p = src_p.reshape(NW * NIG, IG, CHUNK)
    dst_p = dst_p.reshape(NW * NIG, IG, CHUNK)
    xa = jnp.concatenate(
        [x, jnp.ones((N_NODES, 1), jnp.float32),
         jnp.zeros((N_NODES, DA - D - 1), jnp.float32)], axis=1)
    zeros = jnp.zeros((ZROWS, DA), jnp.float32)

    partials = _sc_agg(xa, src_p, dst_p, zeros)
    return _tc_mlp(partials, x, W1, b1.reshape(1, D), W2, b2.reshape(1, D),
                   Wself, bself.reshape(1, D))
